# baseline (speedup 1.0000x reference)
import functools

import jax
import jax.numpy as jnp
from jax import lax
from jax.experimental import pallas as pl
from jax.experimental.pallas import tpu as pltpu
from jax.experimental.pallas import tpu_sc as plsc

_DIM = 128
_SIZE = 262144

_NC = 2
_NS = 16
_NW = _NC * _NS

_ROWS = 8
_NROWB = _DIM // _ROWS
_NCOLH = _NW // _NROWB
_CPW = _SIZE // _NCOLH
_CH = 2048
_NCHUNK = _CPW // _CH
_DEPTH = 2

_mesh = plsc.VectorSubcoreMesh(core_axis_name="c", subcore_axis_name="s")


@functools.partial(
    pl.kernel,
    mesh=_mesh,
    out_type=jax.ShapeDtypeStruct((_DIM, _SIZE), jnp.float32),
    scratch_types=(
        [pltpu.VMEM((_ROWS, _CH), jnp.float32)] * _DEPTH
        + [pltpu.SemaphoreType.DMA] * (2 * _DEPTH)
    ),
)
def _sc_copy(bank_hbm, out_hbm, *scratch):
    bufs = scratch[:_DEPTH]
    in_sems = scratch[_DEPTH:2 * _DEPTH]
    out_sems = scratch[2 * _DEPTH:]

    wid = lax.axis_index("s") * _NC + lax.axis_index("c")
    band = wid % _NROWB
    half = wid // _NROWB
    r0 = band * _ROWS
    c0 = half * _CPW

    def _src(i):
        return bank_hbm.at[pl.ds(r0, _ROWS), pl.ds(c0 + i * _CH, _CH)]

    def _dst(i):
        return out_hbm.at[pl.ds(r0, _ROWS), pl.ds(c0 + i * _CH, _CH)]

    for i in range(_DEPTH):
        pltpu.make_async_copy(_src(i), bufs[i], in_sems[i]).start()
    _LAG = 1
    for i in range(_NCHUNK):
        b = i % _DEPTH
        pltpu.make_async_copy(_src(i), bufs[b], in_sems[b]).wait()
        pltpu.make_async_copy(bufs[b], _dst(i), out_sems[b]).start()
        j = i - _LAG
        if j >= 0 and j + _DEPTH < _NCHUNK:
            bj = j % _DEPTH
            pltpu.make_async_copy(bufs[bj], _dst(j), out_sems[bj]).wait()
            pltpu.make_async_copy(_src(j + _DEPTH), bufs[bj], in_sems[bj]).start()
    for i in range(max(0, _NCHUNK - _DEPTH - _LAG + 1), _NCHUNK):
        b = i % _DEPTH
        pltpu.make_async_copy(bufs[b], _dst(i), out_sems[b]).wait()


def kernel(output, bank):
    return (output, _sc_copy(bank))


# speedup vs baseline: 1.2063x; 1.2063x over previous
import functools

import jax
import jax.numpy as jnp
from jax import lax
from jax.experimental import pallas as pl
from jax.experimental.pallas import tpu as pltpu
from jax.experimental.pallas import tpu_sc as plsc

_DIM = 128
_SIZE = 262144

_NC = 2
_NS = 16
_NW = _NC * _NS

_ROWS = 8
_NROWB = _DIM // _ROWS
_NCOLH = _NW // _NROWB
_CPW = _SIZE // _NCOLH
_CH = 8064
_DEPTH = 2
_OFFS = [i * _CH for i in range(_CPW // _CH)] + [(_CPW // _CH) * _CH]
_LENS = [_CH] * (_CPW // _CH) + [_CPW - (_CPW // _CH) * _CH]
_NCHUNK = len(_OFFS)

_mesh = plsc.VectorSubcoreMesh(core_axis_name="c", subcore_axis_name="s")


@functools.partial(
    pl.kernel,
    mesh=_mesh,
    out_type=jax.ShapeDtypeStruct((_DIM, _SIZE), jnp.float32),
    scratch_types=(
        [pltpu.VMEM((_ROWS, _CH), jnp.float32)] * _DEPTH
        + [pltpu.SemaphoreType.DMA] * (2 * _DEPTH)
    ),
)
def _sc_copy(bank_hbm, out_hbm, *scratch):
    bufs = scratch[:_DEPTH]
    in_sems = scratch[_DEPTH:2 * _DEPTH]
    out_sems = scratch[2 * _DEPTH:]

    wid = lax.axis_index("s") * _NC + lax.axis_index("c")
    band = wid % _NROWB
    half = wid // _NROWB
    r0 = band * _ROWS
    c0 = half * _CPW

    def _src(i):
        return bank_hbm.at[pl.ds(r0, _ROWS), pl.ds(c0 + _OFFS[i], _LENS[i])]

    def _dst(i):
        return out_hbm.at[pl.ds(r0, _ROWS), pl.ds(c0 + _OFFS[i], _LENS[i])]

    def _bufref(b, i):
        if _LENS[i] == _CH:
            return bufs[b]
        return bufs[b].at[:, pl.ds(0, _LENS[i])]

    for i in range(_DEPTH):
        pltpu.make_async_copy(_src(i), _bufref(i, i), in_sems[i]).start()
    _LAG = 1
    for i in range(_NCHUNK):
        b = i % _DEPTH
        pltpu.make_async_copy(_src(i), _bufref(b, i), in_sems[b]).wait()
        pltpu.make_async_copy(_bufref(b, i), _dst(i), out_sems[b]).start()
        j = i - _LAG
        if j >= 0 and j + _DEPTH < _NCHUNK:
            bj = j % _DEPTH
            pltpu.make_async_copy(_bufref(bj, j), _dst(j), out_sems[bj]).wait()
            pltpu.make_async_copy(_src(j + _DEPTH), _bufref(bj, j + _DEPTH),
                                  in_sems[bj]).start()
    for i in range(max(0, _NCHUNK - _DEPTH - _LAG + 1), _NCHUNK):
        b = i % _DEPTH
        pltpu.make_async_copy(_bufref(b, i), _dst(i), out_sems[b]).wait()


def kernel(output, bank):
    return (output, _sc_copy(bank))
